# baseline (speedup 1.0000x reference)
import functools

import jax
import jax.numpy as jnp
from jax import lax
from jax.experimental import pallas as pl
from jax.experimental.pallas import tpu as pltpu
from jax.experimental.pallas import tpu_sc as plsc

D = 64
B = 4096
S = 50
BT = 128

_info = plsc.get_sparse_core_info()
NC, NS = _info.num_cores, _info.num_subcores
NW = NC * NS

_mesh = plsc.VectorSubcoreMesh(core_axis_name="c", subcore_axis_name="s")


def _extract_idx(ids_v, s, idx_dst):
    iota = lax.iota(jnp.int32, 16)
    for g in range(8):
        pos = (iota + (g * 16)) * S + s
        idx_dst[pl.ds(g * 16, 16)] = plsc.load_gather(ids_v, [pos])


def _transpose_block(rows_v, t_v):
    iota = lax.iota(jnp.int32, 16)

    def loads(d):
        dvec = jnp.full((16,), d, jnp.int32)
        return [plsc.load_gather(rows_v, [iota + (g * 16), dvec])
                for g in range(8)]

    def stores(d, vals):
        for g in range(8):
            t_v[d // 8, d % 8, pl.ds(g * 16, 16)] = vals[g]

    prev = loads(0)
    for d in range(1, D):
        cur = loads(d)
        stores(d - 1, prev)
        prev = cur
    stores(D - 1, prev)


@functools.partial(
    pl.kernel,
    mesh=_mesh,
    out_type=jax.ShapeDtypeStruct((S, 8, NW, 8, BT), jnp.float32),
    scratch_types=[
        pltpu.VMEM((BT * S,), jnp.int32),
        pltpu.VMEM((BT,), jnp.int32),
        pltpu.VMEM((BT,), jnp.int32),
        pltpu.VMEM((BT, D), jnp.float32),
        pltpu.VMEM((BT, D), jnp.float32),
        pltpu.VMEM((8, 8, BT), jnp.float32),
        pltpu.VMEM((8, 8, BT), jnp.float32),
        pltpu.SemaphoreType.DMA,
        pltpu.SemaphoreType.DMA,
        pltpu.SemaphoreType.DMA,
        pltpu.SemaphoreType.DMA,
    ],
    compiler_params=pltpu.CompilerParams(
        use_tc_tiling_on_sc=False, needs_layout_passes=False),
)
def _gather_kernel(table_hbm, ids_hbm, out_hbm, ids_v, idx0, idx1,
                   rows0, rows1, t0, t1, sg0, sg1, sw0, sw1):
    w = lax.axis_index("s") * NC + lax.axis_index("c")

    pltpu.sync_copy(ids_hbm.at[pl.ds(w * (BT * S), BT * S)], ids_v)

    def gather_start(idx_v, rows_v, sem):
        return pltpu.async_copy(table_hbm.at[idx_v], rows_v, sem)

    def wb_start(t_v, s, sem):
        return pltpu.async_copy(t_v, out_hbm.at[s, :, w], sem)

    def wb_wait(t_v, s, sem):
        pltpu.make_async_copy(t_v, out_hbm.at[s, :, w], sem).wait()

    def g_wait(idx_v, rows_v, sem):
        pltpu.make_async_copy(table_hbm.at[idx_v], rows_v, sem).wait()

    _extract_idx(ids_v, 0, idx0)
    gather_start(idx0, rows0, sg0)

    def body(m, carry):
        a = 2 * m
        b = a + 1
        _extract_idx(ids_v, b, idx1)
        gather_start(idx1, rows1, sg1)
        g_wait(idx0, rows0, sg0)

        @pl.when(m > 0)
        def _():
            wb_wait(t0, a - 2, sw0)

        _transpose_block(rows0, t0)
        wb_start(t0, a, sw0)

        @pl.when(m < S // 2 - 1)
        def _():
            _extract_idx(ids_v, a + 2, idx0)
            gather_start(idx0, rows0, sg0)

        g_wait(idx1, rows1, sg1)

        @pl.when(m > 0)
        def _():
            wb_wait(t1, b - 2, sw1)

        _transpose_block(rows1, t1)
        wb_start(t1, b, sw1)
        return carry

    lax.fori_loop(0, S // 2, body, 0)
    wb_wait(t0, S - 2, sw0)
    wb_wait(t1, S - 1, sw1)


def kernel(instrument_ids, table):
    ids_flat = instrument_ids.reshape(-1).astype(jnp.int32)
    out5 = _gather_kernel(table, ids_flat)
    return out5.transpose(2, 4, 0, 1, 3).reshape(B, S, D)


# speedup vs baseline: 2.0940x; 2.0940x over previous
import functools

import jax
import jax.numpy as jnp
from jax import lax
from jax.experimental import pallas as pl
from jax.experimental.pallas import tpu as pltpu
from jax.experimental.pallas import tpu_sc as plsc

D = 64
B = 4096
S = 50
BT = 128
SG = 5
NSTEP = S // SG

_info = plsc.get_sparse_core_info()
NC, NS = _info.num_cores, _info.num_subcores
NW = NC * NS

_mesh = plsc.VectorSubcoreMesh(core_axis_name="c", subcore_axis_name="s")


def _extract_idx(ids_v, s0, idx_dst):
    iota = lax.iota(jnp.int32, 16)
    for c in range(SG):
        for g in range(8):
            pos = (iota + (g * 16)) * S + (s0 + c)
            idx_dst[pl.ds(c * BT + g * 16, 16)] = plsc.load_gather(
                ids_v, [pos])


def _transpose_tile(rows_v, j, t_v):
    iota = lax.iota(jnp.int32, 16)
    dtv = [((g * 16) + iota) >> 3 for g in range(4)]
    drv = [((g * 16) + iota) & 7 for g in range(4)]

    def body(bi, carry):
        row0 = j * BT + bi * 8
        for u in range(8):
            blv = jnp.zeros((16,), jnp.int32) + (bi * 8 + u)
            vals = [rows_v[row0 + u, pl.ds(g * 16, 16)] for g in range(4)]
            for g in range(4):
                plsc.store_scatter(t_v, [dtv[g], drv[g], blv], vals[g])
        return carry

    lax.fori_loop(0, BT // 8, body, 0)


@functools.partial(
    pl.kernel,
    mesh=_mesh,
    out_type=jax.ShapeDtypeStruct((S, 8, NW, 8, BT), jnp.float32),
    scratch_types=[
        pltpu.VMEM((BT * S,), jnp.int32),
        pltpu.VMEM((SG * BT,), jnp.int32),
        pltpu.VMEM((SG * BT,), jnp.int32),
        pltpu.VMEM((SG * BT, D), jnp.float32),
        pltpu.VMEM((SG * BT, D), jnp.float32),
        pltpu.VMEM((8, 8, BT + 1), jnp.float32),
        pltpu.VMEM((8, 8, BT + 1), jnp.float32),
        pltpu.SemaphoreType.DMA,
        pltpu.SemaphoreType.DMA,
        pltpu.SemaphoreType.DMA,
        pltpu.SemaphoreType.DMA,
    ],
    compiler_params=pltpu.CompilerParams(
        use_tc_tiling_on_sc=False, needs_layout_passes=False),
)
def _gather_kernel(table_hbm, ids_hbm, out_hbm, ids_v, idx0, idx1,
                   rows0, rows1, t0, t1, sg0, sg1, sw0, sw1):
    w = lax.axis_index("s") * NC + lax.axis_index("c")

    pltpu.sync_copy(ids_hbm.at[pl.ds(w * (BT * S), BT * S)], ids_v)

    ts = (t0, t1)
    sws = (sw0, sw1)

    def gather_start(idx_v, rows_v, sem):
        return pltpu.async_copy(table_hbm.at[idx_v], rows_v, sem)

    def g_wait(idx_v, rows_v, sem):
        pltpu.make_async_copy(table_hbm.at[idx_v], rows_v, sem).wait()

    def wb_start(tpar, s, sem):
        return pltpu.async_copy(
            ts[tpar].at[:, :, pl.ds(0, BT)], out_hbm.at[s, :, w], sem)

    def wb_wait(tpar, s, sem):
        pltpu.make_async_copy(
            ts[tpar].at[:, :, pl.ds(0, BT)], out_hbm.at[s, :, w], sem).wait()

    def do_step(m, p_is_odd, s0, idx_v, rows_v, sg_cur):
        g_wait(idx_v, rows_v, sg_cur)
        for j in range(SG):
            s = s0 + j
            tpar = (j + (1 if p_is_odd else 0)) % 2
            if (j < 2) and not p_is_odd:
                @pl.when(m > 0)
                def _():
                    wb_wait(tpar, s - 2, sws[tpar])
            else:
                wb_wait(tpar, s - 2, sws[tpar])
            _transpose_tile(rows_v, j, ts[tpar])
            wb_start(tpar, s, sws[tpar])

    _extract_idx(ids_v, 0, idx0)
    gather_start(idx0, rows0, sg0)

    def body(m, carry):
        s0 = 2 * m * SG
        _extract_idx(ids_v, s0 + SG, idx1)
        gather_start(idx1, rows1, sg1)
        do_step(m, False, s0, idx0, rows0, sg0)

        @pl.when(m < NSTEP // 2 - 1)
        def _():
            _extract_idx(ids_v, s0 + 2 * SG, idx0)
            gather_start(idx0, rows0, sg0)

        do_step(m, True, s0 + SG, idx1, rows1, sg1)
        return carry

    lax.fori_loop(0, NSTEP // 2, body, 0)
    wb_wait(0, S - 2, sw0)
    wb_wait(1, S - 1, sw1)


def kernel(instrument_ids, table):
    ids_flat = instrument_ids.reshape(-1).astype(jnp.int32)
    out5 = _gather_kernel(table, ids_flat)
    return out5.transpose(2, 4, 0, 1, 3).reshape(B, S, D)
